# Initial kernel scaffold; baseline (speedup 1.0000x reference)
import jax
import jax.numpy as jnp
from jax.experimental import pallas as pl


def kernel(pred, target):
    raise NotImplementedError("write your pallas kernel here")


# speedup vs baseline: 295.2086x; 295.2086x over previous
import jax
import jax.numpy as jnp
from jax.experimental import pallas as pl
from jax.experimental.pallas import tpu as pltpu


def _lovasz_kernel(pred_ref, tgt_ref, out_ref, acc_err, acc_fg, *, nsteps, n_total):
    i = pl.program_id(0)

    @pl.when(i == 0)
    def _init():
        acc_err[...] = jnp.zeros_like(acc_err)
        acc_fg[...] = jnp.zeros_like(acc_fg)

    x = pred_ref[...]
    c_dim = x.shape[0]
    m = jnp.max(x, axis=0, keepdims=True)
    e = jnp.exp(x - m)
    p = e / jnp.sum(e, axis=0, keepdims=True)

    t = tgt_ref[...]
    classes = jax.lax.broadcasted_iota(jnp.int32, (c_dim, 1), 0)
    fg = (t == classes).astype(jnp.float32)
    err = jnp.abs(fg - p)

    acc_err[...] += jnp.sum(err, axis=1, keepdims=True)
    acc_fg[...] += jnp.sum(fg, axis=1, keepdims=True)

    @pl.when(i == nsteps - 1)
    def _fin():
        total = jnp.sum(acc_err[...] * acc_fg[...], keepdims=True)
        out_ref[...] = total / n_total


def kernel(pred, target):
    n, c = pred.shape
    bn = 16384
    nsteps = n // bn

    pred_t = pred.T
    tgt = target.astype(jnp.int32).reshape(1, n)

    import functools
    out = pl.pallas_call(
        functools.partial(_lovasz_kernel, nsteps=nsteps, n_total=float(n)),
        grid=(nsteps,),
        in_specs=[
            pl.BlockSpec((c, bn), lambda i: (0, i)),
            pl.BlockSpec((1, bn), lambda i: (0, i)),
        ],
        out_specs=pl.BlockSpec((1, 1), lambda i: (0, 0)),
        out_shape=jax.ShapeDtypeStruct((1, 1), jnp.float32),
        scratch_shapes=[
            pltpu.VMEM((c, 1), jnp.float32),
            pltpu.VMEM((c, 1), jnp.float32),
        ],
    )(pred_t, tgt)
    return out.reshape(())
